# baseline (speedup 1.0000x reference)
import jax
import jax.numpy as jnp
import numpy as np
from jax.experimental import pallas as pl

TOP_K = 30
NUM_RBF = 16
MAX_REL = 32
_R = 64

_INTERPRET = False

_A_IDX = (0, 2, 3, 4, 1, 1, 1, 1, 0, 0, 0, 4, 4, 3, 0, 2, 3, 4, 2, 3, 4, 2, 3, 2)
_B_IDX = (0, 2, 3, 4, 0, 2, 3, 4, 2, 3, 4, 2, 3, 2, 1, 1, 1, 1, 0, 0, 0, 4, 4, 3)

_BF = jnp.bfloat16
_F32 = jnp.float32


def _split(x):
    hi = x.astype(_BF)
    lo = (x - hi.astype(_F32)).astype(_BF)
    return hi, lo


def _mm(a, b):
    return jnp.dot(a, b, preferred_element_type=_F32)


def _mm_lhs01(a_bf, b):
    bh, bl = _split(b)
    return _mm(a_bf, bh) + _mm(a_bf, bl)


def _mm_rhs01(a, b_bf):
    ah, al = _split(a)
    return _mm(ah, b_bf) + _mm(al, b_bf)


def _selector_consts():
    sa = np.zeros((16, 73), np.float32)
    sb = np.zeros((16, 73), np.float32)
    ss = np.zeros((72, 24), np.float32)
    sa[15, 72] = 1.0
    sb[15, 72] = 1.0
    for p in range(24):
        for c in range(3):
            sa[_A_IDX[p] * 3 + c, p * 3 + c] = 1.0
            sb[_B_IDX[p] * 3 + c, p * 3 + c] = 1.0
            ss[p * 3 + c, p] = 1.0
    smu = np.zeros((26, 400), np.float32)
    mu = np.linspace(2.0, 22.0, NUM_RBF).astype(np.float32)
    for g in range(25):
        for k in range(NUM_RBF):
            smu[g, g * NUM_RBF + k] = 1.0
            smu[25, g * NUM_RBF + k] = -mu[k]
    t3 = np.zeros((3, 234), np.float32)
    s3 = np.zeros((234, 78), np.float32)
    for q in range(78):
        for c in range(3):
            t3[c, 3 * q + c] = 1.0
            s3[3 * q + c, q] = 1.0
    return sa, sb, ss, smu, t3, s3


def _body(x2r, x2f, cat3, mrow, mr, rrow, rc, yr, zt,
          sa, sb, ss, t3, s3, smuh, smul, w1h, w1l, w2h, w2l, b1, gm, bt,
          e_out, eidx_out, cbj_out):
    L = x2f.shape[1]
    R = x2r.shape[1]
    P = R * TOP_K

    def atoms_from_rows(x2):
        n = x2[:, 0:3]
        ca = x2[:, 3:6]
        cc = x2[:, 6:9]
        oo = x2[:, 12:15]
        b = ca - n
        c = cc - ca
        ax = b[:, 1:2] * c[:, 2:3] - b[:, 2:3] * c[:, 1:2]
        ay = b[:, 2:3] * c[:, 0:1] - b[:, 0:1] * c[:, 2:3]
        az = b[:, 0:1] * c[:, 1:2] - b[:, 1:2] * c[:, 0:1]
        a = jnp.concatenate([ax, ay, az], axis=1)
        cb = -0.58273431 * a + 0.56802827 * b - 0.54067466 * c + ca
        return n, ca, cc, oo, cb

    nr, car, ccr, oor, cbr = atoms_from_rows(x2r[0])
    a_table = jnp.concatenate([nr, car, ccr, oor, cbr, rrow[0]], axis=1)

    nf, caf, ccf, oof, cbf = atoms_from_rows(x2f[0])
    table = jnp.concatenate([nf, caf, ccf, oof, cbf, rc[0]], axis=1)

    dx = car[:, 0:1] - cat3[0, 0:1, :]
    dy = car[:, 1:2] - cat3[0, 1:2, :]
    dz = car[:, 2:3] - cat3[0, 2:3, :]
    d_full = jnp.sqrt(dx * dx + dy * dy + dz * dz + 1e-6)
    m2 = mrow[0] * mr[0, 0:1, :]
    dm = m2 * d_full
    dmax = jnp.max(dm, axis=1, keepdims=True)
    dadj = dm + (1.0 - m2) * dmax

    lane_l_f = jax.lax.broadcasted_iota(jnp.int32, (1, L), 1).astype(_F32)
    lane32f = jax.lax.broadcasted_iota(jnp.int32, (R, 32), 1).astype(_F32)
    vals = jnp.zeros((R, 32), _F32)
    idxs = jnp.zeros((R, 32), _F32)
    dw = dadj
    for t in range(TOP_K):
        m = jnp.min(dw, axis=1, keepdims=True)
        sel = jnp.where(dw == m, jnp.broadcast_to(lane_l_f, dw.shape), 1e9)
        idxf = jnp.min(sel, axis=1, keepdims=True)
        vals = jnp.where(lane32f == t, m, vals)
        idxs = jnp.where(lane32f == t, idxf, idxs)
        dw = jnp.where(lane_l_f == idxf, 1e30, dw)

    eidx_out[0] = (idxs[:, :TOP_K] + 0.5).astype(jnp.int32)

    prow = jax.lax.broadcasted_iota(jnp.int32, (P, R), 0)
    pcol = jax.lax.broadcasted_iota(jnp.int32, (P, R), 1)
    rep = jnp.where(prow // TOP_K == pcol, 1.0, 0.0).astype(_BF)
    msel_r = jax.lax.broadcasted_iota(jnp.int32, (P, 32), 0) % TOP_K
    msel_l = jax.lax.broadcasted_iota(jnp.int32, (P, 32), 1)
    msel = jnp.where(msel_r == msel_l, 1.0, 0.0)
    a_pair = _mm_rhs01(a_table, sa[...])
    cat = jnp.concatenate([vals, idxs, a_pair], axis=1)
    big = _mm_lhs01(rep, cat)
    dnb = jnp.sum(big[:, 0:32] * msel, axis=1, keepdims=True)
    eflat = jnp.sum(big[:, 32:64] * msel, axis=1, keepdims=True)
    eidx_i = (eflat + 0.5).astype(jnp.int32)
    pa = big[:, 64:136]
    resid_i = big[:, 136:137]

    lane_li = jax.lax.broadcasted_iota(jnp.int32, (P, L), 1)
    onehot = jnp.where(lane_li == eidx_i, 1.0, 0.0).astype(_BF)
    tb = _mm_rhs01(table, sb[...])
    g = _mm_lhs01(onehot, tb)
    pg = g[:, 0:72]
    resid_g = g[:, 72:73]

    diff = pa - pg
    d2 = _mm_rhs01(diff * diff, ss[...])
    d24 = jnp.sqrt(d2 + 1e-6)

    ones_p = jnp.ones((P, 1), _F32)
    dall = jnp.concatenate([dnb, d24, ones_p], axis=1)
    dh, dl = _split(dall)
    dc = _mm(dh, smuh[...]) + _mm(dh, smul[...]) + _mm(dl, smuh[...])
    z = dc * 0.8
    feats = jnp.exp(-(z * z))

    off = resid_i - resid_g
    dpos = jnp.clip(off + float(MAX_REL), 0.0, float(2 * MAX_REL))
    dpos_i = (dpos + 0.5).astype(jnp.int32)
    lane66 = jax.lax.broadcasted_iota(jnp.int32, (P, 2 * MAX_REL + 2), 1)
    oh66 = jnp.where(lane66 == dpos_i, 1.0, 0.0).astype(_BF)

    fh, fl = _split(feats)
    e_pre = (_mm(fh, w2h[...]) + _mm(fh, w2l[...]) + _mm(fl, w2h[...])
             + _mm(oh66, w1h[...]) + _mm(oh66, w1l[...]) + b1[...])

    mu_e = jnp.mean(e_pre, axis=1, keepdims=True)
    xm = e_pre - mu_e
    var = jnp.mean(xm * xm, axis=1, keepdims=True)
    e_norm = xm / jnp.sqrt(var + 1e-5) * gm[...] + bt[...]
    e_out[0] = e_norm.reshape(R, TOP_K, 128)

    cbrep = _mm_rhs01(cbr, t3[...])
    dyr = yr[0] - cbrep
    d78 = _mm_rhs01(dyr * dyr, s3[...])
    cbx_r, cby_r, cbz_r = cbr[:, 0:1], cbr[:, 1:2], cbr[:, 2:3]
    zx, zy, zz = zt[0, 0:1, :], zt[0, 1:2, :], zt[0, 2:3, :]
    d16 = (cbx_r - zx) ** 2 + (cby_r - zy) ** 2 + (cbz_r - zz) ** 2
    cbj_out[0] = jnp.sqrt(jnp.concatenate([d78, d16], axis=1) + 1e-6)


def kernel(Z, Z_m, Z_t, X, Y, Y_m, mask, atom_mask, residue_idx,
           chain_labels, pos_W, pos_b, edge_W, gamma, beta):
    B, L = X.shape[0], X.shape[1]
    R = _R
    nblk = L // R
    P = R * TOP_K

    x2 = X.reshape(B, L, 15)
    cat3 = jnp.transpose(x2[:, :, 3:6], (0, 2, 1))
    mask_c = mask[..., None]
    mask_r = mask[:, None, :]
    resid_c = residue_idx.astype(jnp.float32)[..., None]
    yr = Y.reshape(B, L, 234)
    ztr = jnp.transpose(Z, (0, 2, 1))

    sa, sb, ss, smu, t3, s3 = _selector_consts()
    sa = jnp.asarray(sa, _BF)
    sb = jnp.asarray(sb, _BF)
    ss = jnp.asarray(ss, _BF)
    t3 = jnp.asarray(t3, _BF)
    s3 = jnp.asarray(s3, _BF)
    smuh, smul = _split(jnp.asarray(smu))
    w1 = jnp.dot(pos_W, edge_W[:16],
                 precision=jax.lax.Precision.HIGHEST)
    b1 = jnp.dot(pos_b[None, :], edge_W[:16],
                 precision=jax.lax.Precision.HIGHEST)
    w1h, w1l = _split(w1)
    w2h, w2l = _split(edge_W[16:])
    gm = gamma[None, :]
    bt = beta[None, :]

    full = lambda shape: pl.BlockSpec(shape, lambda b, r: (0,) * len(shape))
    per_b = lambda shape: pl.BlockSpec(shape, lambda b, r: (b,) + (0,) * (len(shape) - 1))
    per_br = lambda shape: pl.BlockSpec(shape, lambda b, r: (b, r) + (0,) * (len(shape) - 2))

    out_shapes = (
        jax.ShapeDtypeStruct((B, L, TOP_K, 128), jnp.float32),
        jax.ShapeDtypeStruct((B, L, TOP_K), jnp.int32),
        jax.ShapeDtypeStruct((B, L, 94), jnp.float32),
    )
    out_specs = (per_br((1, R, TOP_K, 128)), per_br((1, R, TOP_K)),
                 per_br((1, R, 94)))

    in_specs = [
        per_br((1, R, 15)),
        per_b((1, L, 15)),
        per_b((1, 3, L)),
        per_br((1, R, 1)),
        per_b((1, 1, L)),
        per_br((1, R, 1)),
        per_b((1, L, 1)),
        per_br((1, R, 234)),
        per_b((1, 3, 16)),
        full((16, 73)),
        full((16, 73)),
        full((72, 24)),
        full((3, 234)),
        full((234, 78)),
        full((26, 400)),
        full((26, 400)),
        full((66, 128)),
        full((66, 128)),
        full((400, 128)),
        full((400, 128)),
        full((1, 128)),
        full((1, 128)),
        full((1, 128)),
    ]

    e_full, eidx, cbj = pl.pallas_call(
        _body,
        grid=(B, nblk),
        in_specs=in_specs,
        out_specs=out_specs,
        out_shape=out_shapes,
        interpret=_INTERPRET,
    )(x2, x2, cat3, mask_c, mask_r, resid_c, resid_c, yr, ztr,
      sa, sb, ss, t3, s3, smuh, smul, w1h, w1l, w2h, w2l, b1, gm, bt)

    return e_full, eidx, cbj


# speedup vs baseline: 1.2139x; 1.2139x over previous
import jax
import jax.numpy as jnp
import numpy as np
from jax.experimental import pallas as pl

TOP_K = 30
NUM_RBF = 16
MAX_REL = 32
_R = 128

_INTERPRET = False

_A_IDX = (0, 2, 3, 4, 1, 1, 1, 1, 0, 0, 0, 4, 4, 3, 0, 2, 3, 4, 2, 3, 4, 2, 3, 2)
_B_IDX = (0, 2, 3, 4, 0, 2, 3, 4, 2, 3, 4, 2, 3, 2, 1, 1, 1, 1, 0, 0, 0, 4, 4, 3)

_BF = jnp.bfloat16
_F32 = jnp.float32


def _split(x):
    hi = x.astype(_BF)
    lo = (x - hi.astype(_F32)).astype(_BF)
    return hi, lo


def _mm(a, b):
    return jnp.dot(a, b, preferred_element_type=_F32)


def _mm_lhs01(a_bf, b):
    bh, bl = _split(b)
    return _mm(a_bf, bh) + _mm(a_bf, bl)


def _mm_rhs01(a, b_bf):
    ah, al = _split(a)
    return _mm(ah, b_bf) + _mm(al, b_bf)


def _selector_consts():
    sa = np.zeros((16, 73), np.float32)
    sb = np.zeros((16, 73), np.float32)
    ss = np.zeros((72, 24), np.float32)
    sa[15, 72] = 1.0
    sb[15, 72] = 1.0
    for p in range(24):
        for c in range(3):
            sa[_A_IDX[p] * 3 + c, p * 3 + c] = 1.0
            sb[_B_IDX[p] * 3 + c, p * 3 + c] = 1.0
            ss[p * 3 + c, p] = 1.0
    smu = np.zeros((26, 400), np.float32)
    mu = np.linspace(2.0, 22.0, NUM_RBF).astype(np.float32)
    for g in range(25):
        for k in range(NUM_RBF):
            smu[g, g * NUM_RBF + k] = 1.0
            smu[25, g * NUM_RBF + k] = -mu[k]
    t3 = np.zeros((3, 234), np.float32)
    s3 = np.zeros((234, 78), np.float32)
    for q in range(78):
        for c in range(3):
            t3[c, 3 * q + c] = 1.0
            s3[3 * q + c, q] = 1.0
    return sa, sb, ss, smu, t3, s3


def _body(x2r, x2f, cat3, mrow, mr, rrow, rc, yr, zt,
          sa, sb, ss, t3, s3, smuz, w1h, w1l, w2h, w2l, b1, gm, bt,
          e_out, eidx_out, cbj_out):
    L = x2f.shape[1]
    R = x2r.shape[1]
    P = R * TOP_K

    def atoms_from_rows(x2):
        n = x2[:, 0:3]
        ca = x2[:, 3:6]
        cc = x2[:, 6:9]
        oo = x2[:, 12:15]
        b = ca - n
        c = cc - ca
        ax = b[:, 1:2] * c[:, 2:3] - b[:, 2:3] * c[:, 1:2]
        ay = b[:, 2:3] * c[:, 0:1] - b[:, 0:1] * c[:, 2:3]
        az = b[:, 0:1] * c[:, 1:2] - b[:, 1:2] * c[:, 0:1]
        a = jnp.concatenate([ax, ay, az], axis=1)
        cb = -0.58273431 * a + 0.56802827 * b - 0.54067466 * c + ca
        return n, ca, cc, oo, cb

    nr, car, ccr, oor, cbr = atoms_from_rows(x2r[0])
    a_table = jnp.concatenate([nr, car, ccr, oor, cbr, rrow[0]], axis=1)

    nf, caf, ccf, oof, cbf = atoms_from_rows(x2f[0])
    table = jnp.concatenate([nf, caf, ccf, oof, cbf, rc[0]], axis=1)

    dx = car[:, 0:1] - cat3[0, 0:1, :]
    dy = car[:, 1:2] - cat3[0, 1:2, :]
    dz = car[:, 2:3] - cat3[0, 2:3, :]
    d_full = jnp.sqrt(dx * dx + dy * dy + dz * dz + 1e-6)
    m2 = mrow[0] * mr[0, 0:1, :]
    dm = m2 * d_full
    dmax = jnp.max(dm, axis=1, keepdims=True)
    dadj = dm + (1.0 - m2) * dmax

    lane_l_f = jax.lax.broadcasted_iota(jnp.int32, (1, L), 1).astype(_F32)
    lane32f = jax.lax.broadcasted_iota(jnp.int32, (R, 32), 1).astype(_F32)
    vals = jnp.zeros((R, 32), _F32)
    idxs = jnp.zeros((R, 32), _F32)
    dw = dadj
    for t in range(TOP_K):
        m = jnp.min(dw, axis=1, keepdims=True)
        sel = jnp.where(dw == m, jnp.broadcast_to(lane_l_f, dw.shape), 1e9)
        idxf = jnp.min(sel, axis=1, keepdims=True)
        vals = jnp.where(lane32f == t, m, vals)
        idxs = jnp.where(lane32f == t, idxf, idxs)
        dw = jnp.where(lane_l_f == idxf, 1e30, dw)

    eidx_out[0] = (idxs[:, :TOP_K] + 0.5).astype(jnp.int32)

    prow = jax.lax.broadcasted_iota(jnp.int32, (P, R), 0)
    pcol = jax.lax.broadcasted_iota(jnp.int32, (P, R), 1)
    rep = jnp.where(prow // TOP_K == pcol, 1.0, 0.0).astype(_BF)
    msel_r = jax.lax.broadcasted_iota(jnp.int32, (P, 32), 0) % TOP_K
    msel_l = jax.lax.broadcasted_iota(jnp.int32, (P, 32), 1)
    msel = jnp.where(msel_r == msel_l, 1.0, 0.0)
    a_pair = _mm_rhs01(a_table, sa[...])
    cat = jnp.concatenate([vals, idxs, a_pair], axis=1)
    big = _mm_lhs01(rep, cat)
    dnb = jnp.sum(big[:, 0:32] * msel, axis=1, keepdims=True)
    eflat = jnp.sum(big[:, 32:64] * msel, axis=1, keepdims=True)
    eidx_i = (eflat + 0.5).astype(jnp.int32)
    pa = big[:, 64:136]
    resid_i = big[:, 136:137]

    lane_li = jax.lax.broadcasted_iota(jnp.int32, (P, L), 1)
    onehot = jnp.where(lane_li == eidx_i, 1.0, 0.0).astype(_BF)
    tb = _mm_rhs01(table, sb[...])
    g = _mm_lhs01(onehot, tb)
    pg = g[:, 0:72]
    resid_g = g[:, 72:73]

    diff = pa - pg
    d2 = _mm_rhs01(diff * diff, ss[...])
    d24 = jnp.sqrt(d2 + 1e-6)

    ones_p = jnp.ones((P, 1), _F32)
    dall = jnp.concatenate([dnb, d24, ones_p], axis=1)
    dh, dl = _split(dall)
    lhs3 = jnp.concatenate([dh, dh, dl], axis=1)
    z = _mm(lhs3, smuz[...])
    feats = jnp.exp(-(z * z))

    off = resid_i - resid_g
    dpos = jnp.clip(off + float(MAX_REL), 0.0, float(2 * MAX_REL))
    dpos_i = (dpos + 0.5).astype(jnp.int32)
    lane66 = jax.lax.broadcasted_iota(jnp.int32, (P, 2 * MAX_REL + 2), 1)
    oh66 = jnp.where(lane66 == dpos_i, 1.0, 0.0).astype(_BF)

    fh = feats.astype(_BF)
    e_pre = (_mm(fh, w2h[...]) + _mm(fh, w2l[...])
             + _mm(oh66, w1h[...]) + _mm(oh66, w1l[...]) + b1[...])

    mu_e = jnp.mean(e_pre, axis=1, keepdims=True)
    xm = e_pre - mu_e
    var = jnp.mean(xm * xm, axis=1, keepdims=True)
    e_norm = xm / jnp.sqrt(var + 1e-5) * gm[...] + bt[...]
    e_out[0] = e_norm.reshape(R, TOP_K, 128)

    cbrep = _mm_rhs01(cbr, t3[...])
    dyr = yr[0] - cbrep
    d78 = _mm_rhs01(dyr * dyr, s3[...])
    cbx_r, cby_r, cbz_r = cbr[:, 0:1], cbr[:, 1:2], cbr[:, 2:3]
    zx, zy, zz = zt[0, 0:1, :], zt[0, 1:2, :], zt[0, 2:3, :]
    d16 = (cbx_r - zx) ** 2 + (cby_r - zy) ** 2 + (cbz_r - zz) ** 2
    cbj_out[0] = jnp.sqrt(jnp.concatenate([d78, d16], axis=1) + 1e-6)


def kernel(Z, Z_m, Z_t, X, Y, Y_m, mask, atom_mask, residue_idx,
           chain_labels, pos_W, pos_b, edge_W, gamma, beta):
    B, L = X.shape[0], X.shape[1]
    R = _R
    nblk = L // R
    P = R * TOP_K

    x2 = X.reshape(B, L, 15)
    cat3 = jnp.transpose(x2[:, :, 3:6], (0, 2, 1))
    mask_c = mask[..., None]
    mask_r = mask[:, None, :]
    resid_c = residue_idx.astype(jnp.float32)[..., None]
    yr = Y.reshape(B, L, 234)
    ztr = jnp.transpose(Z, (0, 2, 1))

    sa, sb, ss, smu, t3, s3 = _selector_consts()
    sa = jnp.asarray(sa, _BF)
    sb = jnp.asarray(sb, _BF)
    ss = jnp.asarray(ss, _BF)
    t3 = jnp.asarray(t3, _BF)
    s3 = jnp.asarray(s3, _BF)
    s08h, s08l = _split(jnp.asarray(smu * 0.8))
    smuz = jnp.concatenate([s08h, s08l, s08h], axis=0)
    w1 = jnp.dot(pos_W, edge_W[:16],
                 precision=jax.lax.Precision.HIGHEST)
    b1 = jnp.dot(pos_b[None, :], edge_W[:16],
                 precision=jax.lax.Precision.HIGHEST)
    w1h, w1l = _split(w1)
    w2h, w2l = _split(edge_W[16:])
    gm = gamma[None, :]
    bt = beta[None, :]

    full = lambda shape: pl.BlockSpec(shape, lambda b, r: (0,) * len(shape))
    per_b = lambda shape: pl.BlockSpec(shape, lambda b, r: (b,) + (0,) * (len(shape) - 1))
    per_br = lambda shape: pl.BlockSpec(shape, lambda b, r: (b, r) + (0,) * (len(shape) - 2))

    out_shapes = (
        jax.ShapeDtypeStruct((B, L, TOP_K, 128), jnp.float32),
        jax.ShapeDtypeStruct((B, L, TOP_K), jnp.int32),
        jax.ShapeDtypeStruct((B, L, 94), jnp.float32),
    )
    out_specs = (per_br((1, R, TOP_K, 128)), per_br((1, R, TOP_K)),
                 per_br((1, R, 94)))

    in_specs = [
        per_br((1, R, 15)),
        per_b((1, L, 15)),
        per_b((1, 3, L)),
        per_br((1, R, 1)),
        per_b((1, 1, L)),
        per_br((1, R, 1)),
        per_b((1, L, 1)),
        per_br((1, R, 234)),
        per_b((1, 3, 16)),
        full((16, 73)),
        full((16, 73)),
        full((72, 24)),
        full((3, 234)),
        full((234, 78)),
        full((78, 400)),
        full((66, 128)),
        full((66, 128)),
        full((400, 128)),
        full((400, 128)),
        full((1, 128)),
        full((1, 128)),
        full((1, 128)),
    ]

    e_full, eidx, cbj = pl.pallas_call(
        _body,
        grid=(B, nblk),
        in_specs=in_specs,
        out_specs=out_specs,
        out_shape=out_shapes,
        interpret=_INTERPRET,
    )(x2, x2, cat3, mask_c, mask_r, resid_c, resid_c, yr, ztr,
      sa, sb, ss, t3, s3, smuz, w1h, w1l, w2h, w2l, b1, gm, bt)

    return e_full, eidx, cbj


# speedup vs baseline: 1.2567x; 1.0352x over previous
import jax
import jax.numpy as jnp
import numpy as np
from jax.experimental import pallas as pl

TOP_K = 30
NUM_RBF = 16
MAX_REL = 32
_R = 128

_INTERPRET = False

_A_IDX = (0, 2, 3, 4, 1, 1, 1, 1, 0, 0, 0, 4, 4, 3, 0, 2, 3, 4, 2, 3, 4, 2, 3, 2)
_B_IDX = (0, 2, 3, 4, 0, 2, 3, 4, 2, 3, 4, 2, 3, 2, 1, 1, 1, 1, 0, 0, 0, 4, 4, 3)

_BF = jnp.bfloat16
_F32 = jnp.float32


def _split(x):
    hi = x.astype(_BF)
    lo = (x - hi.astype(_F32)).astype(_BF)
    return hi, lo


def _mm(a, b):
    return jnp.dot(a, b, preferred_element_type=_F32)


def _mm_lhs01(a_bf, b):
    bh, bl = _split(b)
    return _mm(a_bf, bh) + _mm(a_bf, bl)


def _mm_rhs01(a, b_bf):
    ah, al = _split(a)
    return _mm(ah, b_bf) + _mm(al, b_bf)


def _selector_consts():
    sa = np.zeros((16, 73), np.float32)
    sb = np.zeros((16, 73), np.float32)
    ss = np.zeros((72, 24), np.float32)
    sa[15, 72] = 1.0
    sb[15, 72] = 1.0
    for p in range(24):
        for c in range(3):
            sa[_A_IDX[p] * 3 + c, p * 3 + c] = 1.0
            sb[_B_IDX[p] * 3 + c, p * 3 + c] = 1.0
            ss[p * 3 + c, p] = 1.0
    smu = np.zeros((26, 400), np.float32)
    mu = np.linspace(2.0, 22.0, NUM_RBF).astype(np.float32)
    for g in range(25):
        for k in range(NUM_RBF):
            smu[g, g * NUM_RBF + k] = 1.0
            smu[25, g * NUM_RBF + k] = -mu[k]
    t3 = np.zeros((3, 234), np.float32)
    s3 = np.zeros((234, 78), np.float32)
    for q in range(78):
        for c in range(3):
            t3[c, 3 * q + c] = 1.0
            s3[3 * q + c, q] = 1.0
    return sa, sb, ss, smu, t3, s3


def _body(x2r, x2f, cat3, mrow, mr, rrow, rc, yr, zt,
          sa, sb, ss, t3, s3, smuz, w1h, w1l, w2h, w2l, b1, gm, bt,
          e_out, eidx_out, cbj_out):
    L = x2f.shape[1]
    R = x2r.shape[1]
    P = R * TOP_K

    def atoms_from_rows(x2):
        n = x2[:, 0:3]
        ca = x2[:, 3:6]
        cc = x2[:, 6:9]
        oo = x2[:, 12:15]
        b = ca - n
        c = cc - ca
        ax = b[:, 1:2] * c[:, 2:3] - b[:, 2:3] * c[:, 1:2]
        ay = b[:, 2:3] * c[:, 0:1] - b[:, 0:1] * c[:, 2:3]
        az = b[:, 0:1] * c[:, 1:2] - b[:, 1:2] * c[:, 0:1]
        a = jnp.concatenate([ax, ay, az], axis=1)
        cb = -0.58273431 * a + 0.56802827 * b - 0.54067466 * c + ca
        return n, ca, cc, oo, cb

    nr, car, ccr, oor, cbr = atoms_from_rows(x2r[0])
    a_table = jnp.concatenate([nr, car, ccr, oor, cbr, rrow[0]], axis=1)

    nf, caf, ccf, oof, cbf = atoms_from_rows(x2f[0])
    table = jnp.concatenate([nf, caf, ccf, oof, cbf, rc[0]], axis=1)

    dx = car[:, 0:1] - cat3[0, 0:1, :]
    dy = car[:, 1:2] - cat3[0, 1:2, :]
    dz = car[:, 2:3] - cat3[0, 2:3, :]
    d_full = jnp.sqrt(dx * dx + dy * dy + dz * dz + 1e-6)
    m2 = mrow[0] * mr[0, 0:1, :]
    dm = m2 * d_full
    dmax = jnp.max(dm, axis=1, keepdims=True)
    dadj = dm + (1.0 - m2) * dmax

    lane_l_f = jax.lax.broadcasted_iota(jnp.int32, (1, L), 1).astype(_F32)
    lane32f = jax.lax.broadcasted_iota(jnp.int32, (R, 32), 1).astype(_F32)
    vals = jnp.zeros((R, 32), _F32)
    idxs = jnp.zeros((R, 32), _F32)
    dw = dadj
    for t in range(TOP_K):
        m = jnp.min(dw, axis=1, keepdims=True)
        sel = jnp.where(dw == m, jnp.broadcast_to(lane_l_f, dw.shape), 1e9)
        idxf = jnp.min(sel, axis=1, keepdims=True)
        vals = jnp.where(lane32f == t, m, vals)
        idxs = jnp.where(lane32f == t, idxf, idxs)
        dw = jnp.where(lane_l_f == idxf, 1e30, dw)

    eidx_out[0] = (idxs[:, :TOP_K] + 0.5).astype(jnp.int32)

    prow = jax.lax.broadcasted_iota(jnp.int32, (P, R), 0)
    pcol = jax.lax.broadcasted_iota(jnp.int32, (P, R), 1)
    rep = jnp.where(prow // TOP_K == pcol, 1.0, 0.0).astype(_BF)
    msel_r = jax.lax.broadcasted_iota(jnp.int32, (P, 32), 0) % TOP_K
    msel_l = jax.lax.broadcasted_iota(jnp.int32, (P, 32), 1)
    msel = jnp.where(msel_r == msel_l, 1.0, 0.0)
    a_pair = _mm_rhs01(a_table, sa[...])
    aph, apl = _split(a_pair)
    vh, vl = _split(vals)
    ih, il = _split(idxs)
    cat1 = jnp.concatenate(
        [vh, vl, ih, il, aph[:, 0:72], aph[:, 72:73], apl[:, 72:73]],
        axis=1)
    big = _mm(rep, cat1)
    dnb = jnp.sum((big[:, 0:32] + big[:, 32:64]) * msel,
                  axis=1, keepdims=True)
    eflat = jnp.sum((big[:, 64:96] + big[:, 96:128]) * msel,
                    axis=1, keepdims=True)
    eidx_i = (eflat + 0.5).astype(jnp.int32)
    pa = big[:, 128:200]
    resid_i = big[:, 200:201] + big[:, 201:202]

    lane_li = jax.lax.broadcasted_iota(jnp.int32, (P, L), 1)
    onehot = jnp.where(lane_li == eidx_i, 1.0, 0.0).astype(_BF)
    tb = _mm_rhs01(table, sb[...])
    tbh, tbl = _split(tb)
    tb1 = jnp.concatenate(
        [tbh[:, 0:72], tbh[:, 72:73], tbl[:, 72:73]], axis=1)
    g = _mm(onehot, tb1)
    pg = g[:, 0:72]
    resid_g = g[:, 72:73] + g[:, 73:74]

    diff = pa - pg
    d2 = _mm_rhs01(diff * diff, ss[...])
    d24 = jnp.sqrt(d2 + 1e-6)

    ones_p = jnp.ones((P, 1), _F32)
    dall = jnp.concatenate([dnb, d24, ones_p], axis=1)
    dh, dl = _split(dall)
    lhs3 = jnp.concatenate([dh, dh, dl], axis=1)
    z = _mm(lhs3, smuz[...])
    feats = jnp.exp(-(z * z))

    off = resid_i - resid_g
    dpos = jnp.clip(off + float(MAX_REL), 0.0, float(2 * MAX_REL))
    dpos_i = (dpos + 0.5).astype(jnp.int32)
    lane66 = jax.lax.broadcasted_iota(jnp.int32, (P, 2 * MAX_REL + 2), 1)
    oh66 = jnp.where(lane66 == dpos_i, 1.0, 0.0).astype(_BF)

    fh = feats.astype(_BF)
    e_pre = (_mm(fh, w2h[...]) + _mm(fh, w2l[...])
             + _mm(oh66, w1h[...]) + _mm(oh66, w1l[...]) + b1[...])

    mu_e = jnp.mean(e_pre, axis=1, keepdims=True)
    xm = e_pre - mu_e
    var = jnp.mean(xm * xm, axis=1, keepdims=True)
    e_norm = xm / jnp.sqrt(var + 1e-5) * gm[...] + bt[...]
    e_out[0] = e_norm.reshape(R, TOP_K, 128)

    cbrep = _mm_rhs01(cbr, t3[...])
    dyr = yr[0] - cbrep
    d78 = _mm_rhs01(dyr * dyr, s3[...])
    cbx_r, cby_r, cbz_r = cbr[:, 0:1], cbr[:, 1:2], cbr[:, 2:3]
    zx, zy, zz = zt[0, 0:1, :], zt[0, 1:2, :], zt[0, 2:3, :]
    d16 = (cbx_r - zx) ** 2 + (cby_r - zy) ** 2 + (cbz_r - zz) ** 2
    cbj_out[0] = jnp.sqrt(jnp.concatenate([d78, d16], axis=1) + 1e-6)


def kernel(Z, Z_m, Z_t, X, Y, Y_m, mask, atom_mask, residue_idx,
           chain_labels, pos_W, pos_b, edge_W, gamma, beta):
    B, L = X.shape[0], X.shape[1]
    R = _R
    nblk = L // R
    P = R * TOP_K

    x2 = X.reshape(B, L, 15)
    cat3 = jnp.transpose(x2[:, :, 3:6], (0, 2, 1))
    mask_c = mask[..., None]
    mask_r = mask[:, None, :]
    resid_c = residue_idx.astype(jnp.float32)[..., None]
    yr = Y.reshape(B, L, 234)
    ztr = jnp.transpose(Z, (0, 2, 1))

    sa, sb, ss, smu, t3, s3 = _selector_consts()
    sa = jnp.asarray(sa, _BF)
    sb = jnp.asarray(sb, _BF)
    ss = jnp.asarray(ss, _BF)
    t3 = jnp.asarray(t3, _BF)
    s3 = jnp.asarray(s3, _BF)
    s08h, s08l = _split(jnp.asarray(smu * 0.8))
    smuz = jnp.concatenate([s08h, s08l, s08h], axis=0)
    w1 = jnp.dot(pos_W, edge_W[:16],
                 precision=jax.lax.Precision.HIGHEST)
    b1 = jnp.dot(pos_b[None, :], edge_W[:16],
                 precision=jax.lax.Precision.HIGHEST)
    w1h, w1l = _split(w1)
    w2h, w2l = _split(edge_W[16:])
    gm = gamma[None, :]
    bt = beta[None, :]

    full = lambda shape: pl.BlockSpec(shape, lambda b, r: (0,) * len(shape))
    per_b = lambda shape: pl.BlockSpec(shape, lambda b, r: (b,) + (0,) * (len(shape) - 1))
    per_br = lambda shape: pl.BlockSpec(shape, lambda b, r: (b, r) + (0,) * (len(shape) - 2))

    out_shapes = (
        jax.ShapeDtypeStruct((B, L, TOP_K, 128), jnp.float32),
        jax.ShapeDtypeStruct((B, L, TOP_K), jnp.int32),
        jax.ShapeDtypeStruct((B, L, 94), jnp.float32),
    )
    out_specs = (per_br((1, R, TOP_K, 128)), per_br((1, R, TOP_K)),
                 per_br((1, R, 94)))

    in_specs = [
        per_br((1, R, 15)),
        per_b((1, L, 15)),
        per_b((1, 3, L)),
        per_br((1, R, 1)),
        per_b((1, 1, L)),
        per_br((1, R, 1)),
        per_b((1, L, 1)),
        per_br((1, R, 234)),
        per_b((1, 3, 16)),
        full((16, 73)),
        full((16, 73)),
        full((72, 24)),
        full((3, 234)),
        full((234, 78)),
        full((78, 400)),
        full((66, 128)),
        full((66, 128)),
        full((400, 128)),
        full((400, 128)),
        full((1, 128)),
        full((1, 128)),
        full((1, 128)),
    ]

    e_full, eidx, cbj = pl.pallas_call(
        _body,
        grid=(B, nblk),
        in_specs=in_specs,
        out_specs=out_specs,
        out_shape=out_shapes,
        interpret=_INTERPRET,
    )(x2, x2, cat3, mask_c, mask_r, resid_c, resid_c, yr, ztr,
      sa, sb, ss, t3, s3, smuz, w1h, w1l, w2h, w2l, b1, gm, bt)

    return e_full, eidx, cbj
